# baseline (device time: 217219 ns/iter reference)
import jax
import jax.numpy as jnp
from jax import lax
from jax.experimental import pallas as pl
from jax.experimental.pallas import tpu as pltpu

N_DEV = 8
SQ = 2048
HQ = 8
DH = 128
D_MODEL = HQ * DH
SCALE = 0.08838834764831843
BLK = 64
RES = 4

PACK_ROWS = 2176
CH = PACK_ROWS // N_DEV


def _ring_allreduce_body(
    x_ref, out_ref, send_buf, recv_buf, send_sem, recv_sem, credit_sem
):
    my = lax.axis_index("i")
    left = lax.rem(my + N_DEV - 1, N_DEV)
    right = lax.rem(my + 1, N_DEV)

    barrier = pltpu.get_barrier_semaphore()
    for nbr in (left, right):
        pl.semaphore_signal(
            barrier, inc=1, device_id=(nbr,), device_id_type=pl.DeviceIdType.MESH
        )
    pl.semaphore_wait(barrier, 2)

    out_ref[:] = x_ref[:].astype(jnp.float32)

    def make_rdma():
        return pltpu.make_async_remote_copy(
            src_ref=send_buf,
            dst_ref=recv_buf,
            send_sem=send_sem,
            recv_sem=recv_sem,
            device_id=(right,),
            device_id_type=pl.DeviceIdType.MESH,
        )

    for s in range(N_DEV - 1):
        send_idx = lax.rem(my - s + N_DEV, N_DEV)
        recv_idx = lax.rem(my - s - 1 + N_DEV, N_DEV)
        send_buf[:] = out_ref[pl.ds(send_idx * CH, CH), :].astype(jnp.bfloat16)
        if s > 0:
            pl.semaphore_wait(credit_sem, 1)
        rdma = make_rdma()
        rdma.start()
        rdma.wait()
        out_ref[pl.ds(recv_idx * CH, CH), :] += recv_buf[:].astype(jnp.float32)
        pl.semaphore_signal(
            credit_sem, inc=1, device_id=(left,), device_id_type=pl.DeviceIdType.MESH
        )

    own = lax.rem(my + 1, N_DEV)
    send_buf[:] = out_ref[pl.ds(own * CH, CH), :].astype(jnp.bfloat16)
    for s in range(N_DEV - 1):
        recv_idx = lax.rem(my - s + N_DEV, N_DEV)
        pl.semaphore_wait(credit_sem, 1)
        rdma = make_rdma()
        rdma.start()
        rdma.wait()
        out_ref[pl.ds(recv_idx * CH, CH), :] = recv_buf[:].astype(jnp.float32)
        if s < N_DEV - 2:
            send_buf[:] = recv_buf[:]
        pl.semaphore_signal(
            credit_sem, inc=1, device_id=(left,), device_id_type=pl.DeviceIdType.MESH
        )

    pl.semaphore_wait(credit_sem, 1)


def _ring_allreduce(packed):
    return pl.pallas_call(
        _ring_allreduce_body,
        out_shape=jax.ShapeDtypeStruct((PACK_ROWS, D_MODEL), jnp.float32),
        in_specs=[pl.BlockSpec(memory_space=pltpu.VMEM)],
        out_specs=pl.BlockSpec(memory_space=pltpu.VMEM),
        scratch_shapes=[
            pltpu.VMEM((CH, D_MODEL), jnp.bfloat16),
            pltpu.VMEM((CH, D_MODEL), jnp.bfloat16),
            pltpu.SemaphoreType.DMA,
            pltpu.SemaphoreType.DMA,
            pltpu.SemaphoreType.REGULAR,
        ],
        compiler_params=pltpu.CompilerParams(collective_id=0),
    )(packed)


def kernel(x, Wq, K_ext, V_ext, Wo):
    f32 = jnp.float32
    bf16 = jnp.bfloat16

    xb = x[0].astype(bf16)
    Q = jnp.dot(xb, Wq.astype(bf16), preferred_element_type=f32)
    Q = Q.reshape(SQ, HQ, DH).astype(bf16)
    K = K_ext[0].astype(bf16)
    V = V_ext[0].astype(bf16)

    n_qb = SQ // BLK
    n_kb = K.shape[0] // BLK
    Qg = Q.reshape(n_qb // RES, RES, BLK, HQ, DH).transpose(1, 0, 2, 3, 4)
    Kg = K.reshape(n_kb // RES, RES, BLK, HQ, DH).transpose(1, 0, 2, 3, 4)
    Vg = V.reshape(n_kb // RES, RES, BLK, HQ, DH).transpose(1, 0, 2, 3, 4)

    s = jnp.einsum("rgahd,rkbhd->rhgakb", Qg, Kg, preferred_element_type=f32)
    w = jnp.exp(s * SCALE)
    l_part = w.sum(axis=(4, 5))
    o_part = jnp.einsum(
        "rhgakb,rkbhd->rgahd", w.astype(bf16), Vg, preferred_element_type=f32
    )

    o2d = (
        o_part.transpose(1, 0, 2, 3, 4).reshape(SQ, D_MODEL).astype(bf16)
    )
    l2d = (
        l_part.transpose(2, 0, 3, 1).reshape(16, D_MODEL).astype(bf16)
    )
    pad = jnp.zeros((PACK_ROWS - SQ - 16, D_MODEL), bf16)
    packed = jnp.concatenate([o2d, l2d, pad], axis=0)

    red = _ring_allreduce(packed)
    o_sum = red[:SQ].reshape(SQ, HQ, DH)
    l_sum = red[SQ : SQ + 16].reshape(SQ, HQ)

    ctx = (o_sum / l_sum[:, :, None]).astype(bf16).reshape(SQ, D_MODEL)
    out = jnp.dot(ctx, Wo.astype(bf16), preferred_element_type=f32)
    return out[None]


# device time: 172778 ns/iter; 1.2572x vs baseline; 1.2572x over previous
import jax
import jax.numpy as jnp
from jax import lax
from jax.experimental import pallas as pl
from jax.experimental.pallas import tpu as pltpu

N_DEV = 8
SQ = 2048
HQ = 8
DH = 128
D_MODEL = HQ * DH
SCALE = 0.08838834764831843
BLK = 64
RES = 4

QCH = SQ // N_DEV
CH = 272
PACK_ROWS = N_DEV * CH


def _hyper(m):
    return m ^ ((m >> 1) & 1)


def _rs_body(x_ref, out_ref, acc, sb0, rb0, sb1, rb1, sb2, rb2, ssems, rsems):
    my = lax.axis_index("i")
    h = _hyper(my)

    barrier = pltpu.get_barrier_semaphore()
    for k in (2, 1, 0):
        p = _hyper(h ^ (1 << k))
        pl.semaphore_signal(
            barrier, inc=1, device_id=(p,), device_id_type=pl.DeviceIdType.MESH
        )
    pl.semaphore_wait(barrier, 3)

    acc[:] = x_ref[:].astype(jnp.float32)

    lo = h * 0
    bufs = ((sb0, rb0), (sb1, rb1), (sb2, rb2))
    for r, k in enumerate((2, 1, 0)):
        half = 1 << k
        hb = (h >> k) & 1
        keep_lo = lo + hb * half
        send_lo = lo + (1 - hb) * half
        p = _hyper(h ^ (1 << k))
        sbuf, rbuf = bufs[r]
        sbuf[:] = acc[pl.ds(send_lo * CH, half * CH), :].astype(jnp.bfloat16)
        rdma = pltpu.make_async_remote_copy(
            src_ref=sbuf,
            dst_ref=rbuf,
            send_sem=ssems.at[r],
            recv_sem=rsems.at[r],
            device_id=(p,),
            device_id_type=pl.DeviceIdType.MESH,
        )
        rdma.start()
        rdma.wait()
        acc[pl.ds(keep_lo * CH, half * CH), :] += rbuf[:].astype(jnp.float32)
        lo = keep_lo

    out_ref[:] = acc[pl.ds(lo * CH, CH), :]


def _reduce_scatter(packed):
    bf16 = jnp.bfloat16
    return pl.pallas_call(
        _rs_body,
        out_shape=jax.ShapeDtypeStruct((CH, D_MODEL), jnp.float32),
        in_specs=[pl.BlockSpec(memory_space=pltpu.VMEM)],
        out_specs=pl.BlockSpec(memory_space=pltpu.VMEM),
        scratch_shapes=[
            pltpu.VMEM((PACK_ROWS, D_MODEL), jnp.float32),
            pltpu.VMEM((4 * CH, D_MODEL), bf16),
            pltpu.VMEM((4 * CH, D_MODEL), bf16),
            pltpu.VMEM((2 * CH, D_MODEL), bf16),
            pltpu.VMEM((2 * CH, D_MODEL), bf16),
            pltpu.VMEM((CH, D_MODEL), bf16),
            pltpu.VMEM((CH, D_MODEL), bf16),
            pltpu.SemaphoreType.DMA((3,)),
            pltpu.SemaphoreType.DMA((3,)),
        ],
        compiler_params=pltpu.CompilerParams(collective_id=0),
    )(packed)


def _ag_body(y_ref, out_ref, sb0, rb0, sb1, rb1, sb2, rb2, ssems, rsems):
    my = lax.axis_index("i")
    h = _hyper(my)

    barrier = pltpu.get_barrier_semaphore()
    for k in (0, 1, 2):
        p = _hyper(h ^ (1 << k))
        pl.semaphore_signal(
            barrier, inc=1, device_id=(p,), device_id_type=pl.DeviceIdType.MESH
        )
    pl.semaphore_wait(barrier, 3)

    out_ref[pl.ds(h * QCH, QCH), :] = y_ref[:]

    lo = h
    bufs = ((sb0, rb0), (sb1, rb1), (sb2, rb2))
    for r, k in enumerate((0, 1, 2)):
        sz = 1 << r
        p = _hyper(h ^ (1 << k))
        sbuf, rbuf = bufs[r]
        sbuf[:] = out_ref[pl.ds(lo * QCH, sz * QCH), :].astype(jnp.bfloat16)
        rdma = pltpu.make_async_remote_copy(
            src_ref=sbuf,
            dst_ref=rbuf,
            send_sem=ssems.at[r],
            recv_sem=rsems.at[r],
            device_id=(p,),
            device_id_type=pl.DeviceIdType.MESH,
        )
        rdma.start()
        rdma.wait()
        recv_lo = lo ^ (1 << k)
        out_ref[pl.ds(recv_lo * QCH, sz * QCH), :] = rbuf[:].astype(jnp.float32)
        lo = lo & ~(1 << k)


def _all_gather(y_chunk):
    bf16 = jnp.bfloat16
    return pl.pallas_call(
        _ag_body,
        out_shape=jax.ShapeDtypeStruct((SQ, D_MODEL), jnp.float32),
        in_specs=[pl.BlockSpec(memory_space=pltpu.VMEM)],
        out_specs=pl.BlockSpec(memory_space=pltpu.VMEM),
        scratch_shapes=[
            pltpu.VMEM((QCH, D_MODEL), bf16),
            pltpu.VMEM((QCH, D_MODEL), bf16),
            pltpu.VMEM((2 * QCH, D_MODEL), bf16),
            pltpu.VMEM((2 * QCH, D_MODEL), bf16),
            pltpu.VMEM((4 * QCH, D_MODEL), bf16),
            pltpu.VMEM((4 * QCH, D_MODEL), bf16),
            pltpu.SemaphoreType.DMA((3,)),
            pltpu.SemaphoreType.DMA((3,)),
        ],
        compiler_params=pltpu.CompilerParams(collective_id=1),
    )(y_chunk)


def kernel(x, Wq, K_ext, V_ext, Wo):
    f32 = jnp.float32
    bf16 = jnp.bfloat16

    xb = x[0].astype(bf16)
    Q = jnp.dot(xb, Wq.astype(bf16), preferred_element_type=f32)
    Q = Q.reshape(SQ, HQ, DH).astype(bf16)
    K = K_ext[0].astype(bf16)
    V = V_ext[0].astype(bf16)

    n_qb = SQ // BLK
    n_kb = K.shape[0] // BLK
    Qg = Q.reshape(n_qb // RES, RES, BLK, HQ, DH).transpose(1, 0, 2, 3, 4)
    Kg = K.reshape(n_kb // RES, RES, BLK, HQ, DH).transpose(1, 0, 2, 3, 4)
    Vg = V.reshape(n_kb // RES, RES, BLK, HQ, DH).transpose(1, 0, 2, 3, 4)

    s = jnp.einsum("rgahd,rkbhd->rhgakb", Qg, Kg, preferred_element_type=f32)
    w = jnp.exp(s * SCALE)
    l_part = w.sum(axis=(4, 5))
    o_part = jnp.einsum(
        "rhgakb,rkbhd->rgahd", w.astype(bf16), Vg, preferred_element_type=f32
    )

    o3 = (
        o_part.transpose(1, 0, 2, 3, 4)
        .reshape(N_DEV, QCH, D_MODEL)
        .astype(bf16)
    )
    l3 = (
        l_part.transpose(2, 0, 3, 1).reshape(N_DEV, 2, D_MODEL).astype(bf16)
    )
    pad3 = jnp.zeros((N_DEV, CH - QCH - 2, D_MODEL), bf16)
    packed = jnp.concatenate([o3, l3, pad3], axis=1).reshape(PACK_ROWS, D_MODEL)

    chunk = _reduce_scatter(packed)
    o_c = chunk[:QCH].reshape(QCH, HQ, DH)
    l_c = chunk[QCH : QCH + 2].reshape(QCH, HQ)
    ctx = (o_c / l_c[:, :, None]).astype(bf16).reshape(QCH, D_MODEL)
    y = jnp.dot(ctx, Wo.astype(bf16), preferred_element_type=f32)

    out = _all_gather(y)
    return out[None]


# device time: 133455 ns/iter; 1.6277x vs baseline; 1.2947x over previous
import jax
import jax.numpy as jnp
from jax import lax
from jax.experimental import pallas as pl
from jax.experimental.pallas import tpu as pltpu

N_DEV = 8
SQ = 2048
HQ = 8
DH = 128
D_MODEL = HQ * DH
SCALE = 0.08838834764831843
BLK = 64
RES = 4

QCH = SQ // N_DEV
CH = 272
PACK_ROWS = N_DEV * CH


HW = D_MODEL // 2


def _hyper(m):
    return m ^ ((m >> 1) & 1)


def _bit(v, k):
    return (v >> k) & 1


def _c_rs_b(pos):
    return 4 * _bit(pos, 0) + 2 * _bit(pos, 2) + _bit(pos, 1)


def _c_ag_b(pos):
    return 4 * _bit(pos, 1) + 2 * _bit(pos, 0) + _bit(pos, 2)


def _rdma(sbuf, rbuf, ssem, rsem, p):
    return pltpu.make_async_remote_copy(
        src_ref=sbuf,
        dst_ref=rbuf,
        send_sem=ssem,
        recv_sem=rsem,
        device_id=(p,),
        device_id_type=pl.DeviceIdType.MESH,
    )


def _neighbor_barrier(h):
    barrier = pltpu.get_barrier_semaphore()
    for k in (0, 1, 2):
        p = _hyper(h ^ (1 << k))
        pl.semaphore_signal(
            barrier, inc=1, device_id=(p,), device_id_type=pl.DeviceIdType.MESH
        )
    pl.semaphore_wait(barrier, 3)


def _rs_body(
    x_ref, out_ref, acc,
    sa0, ra0, sa1, ra1, sa2, ra2,
    sb0, rb0, sb1, rb1, sb2, rb2,
    ssa, rsa, ssb, rsb,
):
    f32, bf16 = jnp.float32, jnp.bfloat16
    my = lax.axis_index("i")
    h = _hyper(my)
    _neighbor_barrier(h)

    acc[:] = x_ref[:].astype(f32)

    loA = h * 0
    loB = h * 0
    bufsA = ((sa0, ra0), (sa1, ra1), (sa2, ra2))
    bufsB = ((sb0, rb0), (sb1, rb1), (sb2, rb2))
    for r, (kA, kB) in enumerate(((2, 1), (1, 0), (0, 2))):
        half = 4 >> r
        sbufA, rbufA = bufsA[r]
        sbufB, rbufB = bufsB[r]

        hbA = _bit(h, kA)
        keepA = loA + hbA * half
        sendA = loA + (1 - hbA) * half
        sbufA[:] = acc[pl.ds(sendA * CH, half * CH), :HW].astype(bf16)
        rdA = _rdma(sbufA, rbufA, ssa.at[r], rsa.at[r], _hyper(h ^ (1 << kA)))
        rdA.start()

        hbB = _bit(h, kB)
        keepB = loB + hbB * half
        sendB = loB + (1 - hbB) * half
        for j in range(half):
            c = _c_rs_b(sendB + j)
            sbufB[j * CH : (j + 1) * CH, :] = acc[
                pl.ds(c * CH, CH), HW:
            ].astype(bf16)
        rdB = _rdma(sbufB, rbufB, ssb.at[r], rsb.at[r], _hyper(h ^ (1 << kB)))
        rdB.start()

        rdA.wait()
        acc[pl.ds(keepA * CH, half * CH), :HW] += rbufA[:].astype(f32)
        rdB.wait()
        for j in range(half):
            c = _c_rs_b(keepB + j)
            acc[pl.ds(c * CH, CH), HW:] += rbufB[j * CH : (j + 1) * CH, :].astype(
                f32
            )
        loA, loB = keepA, keepB

    out_ref[:] = acc[pl.ds(h * CH, CH), :]


def _reduce_scatter(packed):
    f32, bf16 = jnp.float32, jnp.bfloat16
    bufs = []
    for half in (4, 2, 1):
        bufs += [pltpu.VMEM((half * CH, HW), bf16)] * 2
    for half in (4, 2, 1):
        bufs += [pltpu.VMEM((half * CH, HW), bf16)] * 2
    return pl.pallas_call(
        _rs_body,
        out_shape=jax.ShapeDtypeStruct((CH, D_MODEL), f32),
        in_specs=[pl.BlockSpec(memory_space=pltpu.VMEM)],
        out_specs=pl.BlockSpec(memory_space=pltpu.VMEM),
        scratch_shapes=[pltpu.VMEM((PACK_ROWS, D_MODEL), f32)]
        + bufs
        + [
            pltpu.SemaphoreType.DMA((3,)),
            pltpu.SemaphoreType.DMA((3,)),
            pltpu.SemaphoreType.DMA((3,)),
            pltpu.SemaphoreType.DMA((3,)),
        ],
        compiler_params=pltpu.CompilerParams(collective_id=0),
    )(packed)


def _ag_body(
    y_ref, out_ref,
    sa0, ra0, sa1, ra1, sa2, ra2,
    sb0, rb0, sb1, rb1, sb2, rb2,
    ssa, rsa, ssb, rsb,
):
    f32, bf16 = jnp.float32, jnp.bfloat16
    my = lax.axis_index("i")
    h = _hyper(my)
    _neighbor_barrier(h)

    out_ref[pl.ds(h * QCH, QCH), :] = y_ref[:]

    loA = h
    vB = _bit(h, 1) + 2 * _bit(h, 2) + 4 * _bit(h, 0)
    bufsA = ((sa0, ra0), (sa1, ra1), (sa2, ra2))
    bufsB = ((sb0, rb0), (sb1, rb1), (sb2, rb2))
    for r, (kA, kB) in enumerate(((0, 1), (1, 2), (2, 0))):
        sz = 1 << r
        sbufA, rbufA = bufsA[r]
        sbufB, rbufB = bufsB[r]

        sbufA[:] = out_ref[pl.ds(loA * QCH, sz * QCH), :HW].astype(bf16)
        rdA = _rdma(sbufA, rbufA, ssa.at[r], rsa.at[r], _hyper(h ^ (1 << kA)))
        rdA.start()

        for j in range(sz):
            c = _c_ag_b(vB + j)
            sbufB[j * QCH : (j + 1) * QCH, :] = out_ref[
                pl.ds(c * QCH, QCH), HW:
            ].astype(bf16)
        rdB = _rdma(sbufB, rbufB, ssb.at[r], rsb.at[r], _hyper(h ^ (1 << kB)))
        rdB.start()

        rdA.wait()
        recv_loA = loA ^ (1 << kA)
        out_ref[pl.ds(recv_loA * QCH, sz * QCH), :HW] = rbufA[:].astype(f32)
        rdB.wait()
        vB_recv = vB ^ sz
        for j in range(sz):
            c = _c_ag_b(vB_recv + j)
            out_ref[pl.ds(c * QCH, QCH), HW:] = rbufB[
                j * QCH : (j + 1) * QCH, :
            ].astype(f32)
        loA = loA & ~(1 << kA)
        vB = vB & ~sz


def _all_gather(y_chunk):
    f32, bf16 = jnp.float32, jnp.bfloat16
    bufs = []
    for sz in (1, 2, 4):
        bufs += [pltpu.VMEM((sz * QCH, HW), bf16)] * 2
    for sz in (1, 2, 4):
        bufs += [pltpu.VMEM((sz * QCH, HW), bf16)] * 2
    return pl.pallas_call(
        _ag_body,
        out_shape=jax.ShapeDtypeStruct((SQ, D_MODEL), f32),
        in_specs=[pl.BlockSpec(memory_space=pltpu.VMEM)],
        out_specs=pl.BlockSpec(memory_space=pltpu.VMEM),
        scratch_shapes=bufs
        + [
            pltpu.SemaphoreType.DMA((3,)),
            pltpu.SemaphoreType.DMA((3,)),
            pltpu.SemaphoreType.DMA((3,)),
            pltpu.SemaphoreType.DMA((3,)),
        ],
        compiler_params=pltpu.CompilerParams(collective_id=1),
    )(y_chunk)


def kernel(x, Wq, K_ext, V_ext, Wo):
    f32 = jnp.float32
    bf16 = jnp.bfloat16

    xb = x[0].astype(bf16)
    Q = jnp.dot(xb, Wq.astype(bf16), preferred_element_type=f32)
    Q = Q.reshape(SQ, HQ, DH).astype(bf16)
    K = K_ext[0].astype(bf16)
    V = V_ext[0].astype(bf16)

    n_qb = SQ // BLK
    n_kb = K.shape[0] // BLK
    Qg = Q.reshape(n_qb // RES, RES, BLK, HQ, DH).transpose(1, 0, 2, 3, 4)
    Kg = K.reshape(n_kb // RES, RES, BLK, HQ, DH).transpose(1, 0, 2, 3, 4)
    Vg = V.reshape(n_kb // RES, RES, BLK, HQ, DH).transpose(1, 0, 2, 3, 4)

    s = jnp.einsum("rgahd,rkbhd->rhgakb", Qg, Kg, preferred_element_type=f32)
    w = jnp.exp(s * SCALE)
    l_part = w.sum(axis=(4, 5))
    o_part = jnp.einsum(
        "rhgakb,rkbhd->rgahd", w.astype(bf16), Vg, preferred_element_type=f32
    )

    o3 = (
        o_part.transpose(1, 0, 2, 3, 4)
        .reshape(N_DEV, QCH, D_MODEL)
        .astype(bf16)
    )
    l3 = (
        l_part.transpose(2, 0, 3, 1).reshape(N_DEV, 2, D_MODEL).astype(bf16)
    )
    pad3 = jnp.zeros((N_DEV, CH - QCH - 2, D_MODEL), bf16)
    packed = jnp.concatenate([o3, l3, pad3], axis=1).reshape(PACK_ROWS, D_MODEL)

    chunk = _reduce_scatter(packed)
    o_c = chunk[:QCH].reshape(QCH, HQ, DH)
    l_c = chunk[QCH : QCH + 2].reshape(QCH, HQ)
    ctx = (o_c / l_c[:, :, None]).astype(bf16).reshape(QCH, D_MODEL)
    y = jnp.dot(ctx, Wo.astype(bf16), preferred_element_type=f32)

    out = _all_gather(y)
    return out[None]


# device time: 122231 ns/iter; 1.7771x vs baseline; 1.0918x over previous
import jax
import jax.numpy as jnp
from jax import lax
from jax.experimental import pallas as pl
from jax.experimental.pallas import tpu as pltpu

N_DEV = 8
SQ = 2048
HQ = 8
DH = 128
D_MODEL = HQ * DH
SCALE = 0.08838834764831843
BLK = 64
RES = 4

QCH = SQ // N_DEV
CH = 272
PACK_ROWS = N_DEV * CH


HW = D_MODEL // 2


def _hyper(m):
    return m ^ ((m >> 1) & 1)


def _bit(v, k):
    return (v >> k) & 1


def _c_rs_b(pos):
    return 4 * _bit(pos, 0) + 2 * _bit(pos, 2) + _bit(pos, 1)


def _c_ag_b(pos):
    return 4 * _bit(pos, 1) + 2 * _bit(pos, 0) + _bit(pos, 2)


def _rdma(sbuf, rbuf, ssem, rsem, p):
    return pltpu.make_async_remote_copy(
        src_ref=sbuf,
        dst_ref=rbuf,
        send_sem=ssem,
        recv_sem=rsem,
        device_id=(p,),
        device_id_type=pl.DeviceIdType.MESH,
    )


def _neighbor_barrier(h):
    barrier = pltpu.get_barrier_semaphore()
    for k in (0, 1, 2):
        p = _hyper(h ^ (1 << k))
        pl.semaphore_signal(
            barrier, inc=1, device_id=(p,), device_id_type=pl.DeviceIdType.MESH
        )
    pl.semaphore_wait(barrier, 3)


def _rs_body(
    x_ref, out_ref, acc,
    sa0, ra0, sa1, ra1, sa2, ra2,
    sb0, rb0, sb1, rb1, sb2, rb2,
    ssa, rsa, ssb, rsb,
):
    f32, bf16 = jnp.float32, jnp.bfloat16
    my = lax.axis_index("i")
    h = _hyper(my)
    _neighbor_barrier(h)

    acc[:] = x_ref[:].astype(f32)

    loA = h * 0
    loB = h * 0
    bufsA = ((sa0, ra0), (sa1, ra1), (sa2, ra2))
    bufsB = ((sb0, rb0), (sb1, rb1), (sb2, rb2))
    for r, (kA, kB) in enumerate(((2, 1), (1, 0), (0, 2))):
        half = 4 >> r
        sbufA, rbufA = bufsA[r]
        sbufB, rbufB = bufsB[r]

        hbA = _bit(h, kA)
        keepA = loA + hbA * half
        sendA = loA + (1 - hbA) * half
        sbufA[:] = acc[pl.ds(sendA * CH, half * CH), :HW].astype(bf16)
        rdA = _rdma(sbufA, rbufA, ssa.at[r], rsa.at[r], _hyper(h ^ (1 << kA)))
        rdA.start()

        hbB = _bit(h, kB)
        keepB = loB + hbB * half
        sendB = loB + (1 - hbB) * half
        for j in range(half):
            c = _c_rs_b(sendB + j)
            sbufB[j * CH : (j + 1) * CH, :] = acc[
                pl.ds(c * CH, CH), HW:
            ].astype(bf16)
        rdB = _rdma(sbufB, rbufB, ssb.at[r], rsb.at[r], _hyper(h ^ (1 << kB)))
        rdB.start()

        rdA.wait()
        acc[pl.ds(keepA * CH, half * CH), :HW] += rbufA[:].astype(f32)
        rdB.wait()
        for j in range(half):
            c = _c_rs_b(keepB + j)
            acc[pl.ds(c * CH, CH), HW:] += rbufB[j * CH : (j + 1) * CH, :].astype(
                f32
            )
        loA, loB = keepA, keepB

    out_ref[:] = acc[pl.ds(h * CH, CH), :]


def _reduce_scatter(packed):
    f32, bf16 = jnp.float32, jnp.bfloat16
    bufs = []
    for half in (4, 2, 1):
        bufs += [pltpu.VMEM((half * CH, HW), bf16)] * 2
    for half in (4, 2, 1):
        bufs += [pltpu.VMEM((half * CH, HW), bf16)] * 2
    return pl.pallas_call(
        _rs_body,
        out_shape=jax.ShapeDtypeStruct((CH, D_MODEL), f32),
        in_specs=[pl.BlockSpec(memory_space=pltpu.VMEM)],
        out_specs=pl.BlockSpec(memory_space=pltpu.VMEM),
        scratch_shapes=[pltpu.VMEM((PACK_ROWS, D_MODEL), f32)]
        + bufs
        + [
            pltpu.SemaphoreType.DMA((3,)),
            pltpu.SemaphoreType.DMA((3,)),
            pltpu.SemaphoreType.DMA((3,)),
            pltpu.SemaphoreType.DMA((3,)),
        ],
        compiler_params=pltpu.CompilerParams(collective_id=0),
    )(packed)


def _ag_body(
    y_ref, out_ref,
    sa0, ra0, sa1, ra1, sa2, ra2,
    sb0, rb0, sb1, rb1, sb2, rb2,
    ssa, rsa, ssb, rsb,
):
    f32, bf16 = jnp.float32, jnp.bfloat16
    my = lax.axis_index("i")
    h = _hyper(my)
    _neighbor_barrier(h)

    out_ref[pl.ds(h * QCH, QCH), :] = y_ref[:]

    loA = h
    vB = _bit(h, 1) + 2 * _bit(h, 2) + 4 * _bit(h, 0)
    bufsA = ((sa0, ra0), (sa1, ra1), (sa2, ra2))
    bufsB = ((sb0, rb0), (sb1, rb1), (sb2, rb2))
    for r, (kA, kB) in enumerate(((0, 1), (1, 2), (2, 0))):
        sz = 1 << r
        sbufA, rbufA = bufsA[r]
        sbufB, rbufB = bufsB[r]

        sbufA[:] = out_ref[pl.ds(loA * QCH, sz * QCH), :HW].astype(bf16)
        rdA = _rdma(sbufA, rbufA, ssa.at[r], rsa.at[r], _hyper(h ^ (1 << kA)))
        rdA.start()

        for j in range(sz):
            c = _c_ag_b(vB + j)
            sbufB[j * QCH : (j + 1) * QCH, :] = out_ref[
                pl.ds(c * QCH, QCH), HW:
            ].astype(bf16)
        rdB = _rdma(sbufB, rbufB, ssb.at[r], rsb.at[r], _hyper(h ^ (1 << kB)))
        rdB.start()

        rdA.wait()
        recv_loA = loA ^ (1 << kA)
        out_ref[pl.ds(recv_loA * QCH, sz * QCH), :HW] = rbufA[:].astype(f32)
        rdB.wait()
        vB_recv = vB ^ sz
        for j in range(sz):
            c = _c_ag_b(vB_recv + j)
            out_ref[pl.ds(c * QCH, QCH), HW:] = rbufB[
                j * QCH : (j + 1) * QCH, :
            ].astype(f32)
        loA = loA & ~(1 << kA)
        vB = vB & ~sz


def _all_gather(y_chunk):
    f32, bf16 = jnp.float32, jnp.bfloat16
    bufs = []
    for sz in (1, 2, 4):
        bufs += [pltpu.VMEM((sz * QCH, HW), bf16)] * 2
    for sz in (1, 2, 4):
        bufs += [pltpu.VMEM((sz * QCH, HW), bf16)] * 2
    return pl.pallas_call(
        _ag_body,
        out_shape=jax.ShapeDtypeStruct((SQ, D_MODEL), f32),
        in_specs=[pl.BlockSpec(memory_space=pltpu.VMEM)],
        out_specs=pl.BlockSpec(memory_space=pltpu.VMEM),
        scratch_shapes=bufs
        + [
            pltpu.SemaphoreType.DMA((3,)),
            pltpu.SemaphoreType.DMA((3,)),
            pltpu.SemaphoreType.DMA((3,)),
            pltpu.SemaphoreType.DMA((3,)),
        ],
        compiler_params=pltpu.CompilerParams(collective_id=1),
    )(y_chunk)


def _attn_body(xg_ref, wq_ref, kg_ref, vg_ref, o_ref, l_ref):
    f32, bf16 = jnp.float32, jnp.bfloat16
    xr = xg_ref[0]
    wq = wq_ref[...]
    q = jax.lax.dot(xr, wq, preferred_element_type=f32).astype(bf16)
    k = kg_ref[0, 0]
    s = jax.lax.dot_general(
        q, k, (((1,), (1,)), ((), ())), preferred_element_type=f32
    )
    w = jnp.exp(s * SCALE)
    l_ref[0, 0] = w.sum(axis=1).reshape(4, 128)
    o = jax.lax.dot(
        w.astype(bf16), vg_ref[0, 0], preferred_element_type=f32
    )
    o_ref[:, 0, :, :] = o.astype(bf16).reshape(8, BLK, DH)


def _attention(xg, Wqb, Kg, Vg):
    f32, bf16 = jnp.float32, jnp.bfloat16
    o_sd = jax.ShapeDtypeStruct((N_DEV, RES, BLK, D_MODEL), bf16)
    l_sd = jax.ShapeDtypeStruct((RES, HQ, 4, 128), f32)
    return pl.pallas_call(
        _attn_body,
        grid=(RES, HQ),
        in_specs=[
            pl.BlockSpec((1, 512, D_MODEL), lambda r, h: (r, 0, 0)),
            pl.BlockSpec((D_MODEL, DH), lambda r, h: (0, h)),
            pl.BlockSpec((1, 1, 512, DH), lambda r, h: (r, h, 0, 0)),
            pl.BlockSpec((1, 1, 512, DH), lambda r, h: (r, h, 0, 0)),
        ],
        out_shape=[o_sd, l_sd],
        out_specs=[
            pl.BlockSpec((N_DEV, 1, BLK, DH), lambda r, h: (0, r, 0, h)),
            pl.BlockSpec((1, 1, 4, 128), lambda r, h: (r, h, 0, 0)),
        ],
        compiler_params=pltpu.CompilerParams(
            dimension_semantics=("parallel", "parallel")
        ),
    )(xg, Wqb, Kg, Vg)


def kernel(x, Wq, K_ext, V_ext, Wo):
    f32 = jnp.float32
    bf16 = jnp.bfloat16

    xb = x[0].astype(bf16)
    K = K_ext[0].astype(bf16)
    V = V_ext[0].astype(bf16)

    xg = xb.reshape(8, RES, BLK, D_MODEL).transpose(1, 0, 2, 3).reshape(
        RES, 512, D_MODEL
    )
    Kg = K.reshape(8, RES, BLK, HQ, DH).transpose(1, 3, 0, 2, 4).reshape(
        RES, HQ, 512, DH
    )
    Vg = V.reshape(8, RES, BLK, HQ, DH).transpose(1, 3, 0, 2, 4).reshape(
        RES, HQ, 512, DH
    )

    o5, l_part = _attention(xg, Wq.astype(bf16), Kg, Vg)

    o3 = o5.reshape(N_DEV, QCH, D_MODEL)
    l3 = (
        l_part.reshape(RES, HQ, 512)
        .reshape(RES, HQ, 8, BLK)
        .transpose(2, 0, 3, 1)
        .reshape(N_DEV, 2, D_MODEL)
        .astype(bf16)
    )
    pad3 = jnp.zeros((N_DEV, CH - QCH - 2, D_MODEL), bf16)
    packed = jnp.concatenate([o3, l3, pad3], axis=1).reshape(PACK_ROWS, D_MODEL)

    chunk = _reduce_scatter(packed)
    o_c = chunk[:QCH].reshape(QCH, HQ, DH)
    l_c = chunk[QCH : QCH + 2].reshape(QCH, HQ)
    ctx = (o_c / l_c[:, :, None]).astype(bf16).reshape(QCH, D_MODEL)
    y = jnp.dot(ctx, Wo.astype(bf16), preferred_element_type=f32)

    out = _all_gather(y)
    return out[None]
